# speedup vs baseline: 2.7735x; 2.7735x over previous
import jax
import jax.numpy as jnp
from jax import lax
from jax.experimental import pallas as pl
from jax.experimental.pallas import tpu as pltpu
from jax.experimental.pallas import tpu_sc as plsc

_N_ROWS = 600000
_D = 128
_CHUNK = 320
_N_CHUNKS = _N_ROWS // _CHUNK
_NC = 2
_NS = 16
_NW = _NC * _NS
_K_PER_W = -(-_N_CHUNKS // _NW)


def _gather_body(nodes_hbm, idx_hbm, out_hbm, idx_v, rows_v, gsem):
    wid = lax.axis_index("s") * _NC + lax.axis_index("c")

    def chunk(k, carry):
        g = wid + k * _NW

        @pl.when(g < _N_CHUNKS)
        def _():
            base = g * _CHUNK
            pltpu.sync_copy(idx_hbm.at[pl.ds(base, _CHUNK)], idx_v)
            pltpu.async_copy(nodes_hbm.at[idx_v], rows_v, gsem).wait()
            pltpu.sync_copy(rows_v, out_hbm.at[pl.ds(base, _CHUNK)])

        return carry

    lax.fori_loop(0, _K_PER_W, chunk, 0)


@jax.jit
def kernel(nodes, triangles_indexes):
    t, v = triangles_indexes.shape
    idx = triangles_indexes.astype(jnp.int32).reshape(-1)
    mesh = plsc.VectorSubcoreMesh(core_axis_name="c", subcore_axis_name="s")
    gather = pl.kernel(
        _gather_body,
        out_type=jax.ShapeDtypeStruct((_N_ROWS, _D), jnp.float32),
        mesh=mesh,
        scratch_types=[
            pltpu.VMEM((_CHUNK,), jnp.int32),
            pltpu.VMEM((_CHUNK, _D), jnp.float32),
            pltpu.SemaphoreType.DMA,
        ],
    )
    out = gather(nodes, idx)
    return out.reshape(t, v, _D)


# speedup vs baseline: 2.9166x; 1.0516x over previous
import jax
import jax.numpy as jnp
from jax import lax
from jax.experimental import pallas as pl
from jax.experimental.pallas import tpu as pltpu
from jax.experimental.pallas import tpu_sc as plsc

_N_ROWS = 600000
_D = 128
_CHUNK = 480
_N_CHUNKS = _N_ROWS // _CHUNK
_NC = 2
_NS = 16
_NW = _NC * _NS
_K_PER_W = -(-_N_CHUNKS // _NW)


def _gather_body(nodes_hbm, idx_hbm, out_hbm,
                 idx0, idx1, rows0, rows1, g0, g1, s0, s1):
    wid = lax.axis_index("s") * _NC + lax.axis_index("c")
    idx_v = (idx0, idx1)
    rows_v = (rows0, rows1)
    gsem = (g0, g1)
    ssem = (s0, s1)

    def start_gather(k, b):
        base = (wid + k * _NW) * _CHUNK
        pltpu.sync_copy(idx_hbm.at[pl.ds(base, _CHUNK)], idx_v[b])
        pltpu.async_copy(nodes_hbm.at[idx_v[b]], rows_v[b], gsem[b])

    def wait_scatter(b):
        pltpu.make_async_copy(
            rows_v[b], out_hbm.at[pl.ds(0, _CHUNK)], ssem[b]).wait()

    start_gather(0, 0)

    def step(k, b):
        g_k = wid + k * _NW
        g_n = g_k + _NW

        @pl.when(jnp.logical_and(k >= 1, g_n < _N_CHUNKS))
        def _():
            wait_scatter(1 - b)

        @pl.when(g_n < _N_CHUNKS)
        def _():
            start_gather(k + 1, 1 - b)

        @pl.when(g_k < _N_CHUNKS)
        def _():
            pltpu.make_async_copy(
                nodes_hbm.at[idx_v[b]], rows_v[b], gsem[b]).wait()
            pltpu.async_copy(
                rows_v[b], out_hbm.at[pl.ds(g_k * _CHUNK, _CHUNK)], ssem[b])

    def pair(p, carry):
        step(2 * p, 0)
        step(2 * p + 1, 1)
        return carry

    lax.fori_loop(0, _K_PER_W // 2, pair, 0)
    wait_scatter(0)
    wait_scatter(1)


@jax.jit
def kernel(nodes, triangles_indexes):
    t, v = triangles_indexes.shape
    idx = triangles_indexes.astype(jnp.int32).reshape(-1)
    mesh = plsc.VectorSubcoreMesh(core_axis_name="c", subcore_axis_name="s")
    gather = pl.kernel(
        _gather_body,
        out_type=jax.ShapeDtypeStruct((_N_ROWS, _D), jnp.float32),
        mesh=mesh,
        scratch_types=[
            pltpu.VMEM((_CHUNK,), jnp.int32),
            pltpu.VMEM((_CHUNK,), jnp.int32),
            pltpu.VMEM((_CHUNK, _D), jnp.float32),
            pltpu.VMEM((_CHUNK, _D), jnp.float32),
            pltpu.SemaphoreType.DMA,
            pltpu.SemaphoreType.DMA,
            pltpu.SemaphoreType.DMA,
            pltpu.SemaphoreType.DMA,
        ],
    )
    out = gather(nodes, idx)
    return out.reshape(t, v, _D)


# speedup vs baseline: 5.4884x; 1.8818x over previous
import jax
import jax.numpy as jnp
from jax import lax
from jax.experimental import pallas as pl
from jax.experimental.pallas import tpu as pltpu
from jax.experimental.pallas import tpu_sc as plsc

_T = 200000
_D = 128
_TRI = 128
_ROWS = 3 * _TRI
_N_FULL = _T // _TRI
_TAIL_TRI = _T - _N_FULL * _TRI
_TAIL_ROWS = 3 * _TAIL_TRI
_NC = 2
_NS = 16
_NW = _NC * _NS
_K_PER_W = -(-_N_FULL // _NW)
_TAIL_WID = _N_FULL % _NW


def _gather_body(nodes_hbm, idx_hbm, out_hbm, idx_v, rows_v, ti_v, tr_v, gsem):
    wid = lax.axis_index("s") * _NC + lax.axis_index("c")

    def chunk(k, carry):
        g = wid + k * _NW

        @pl.when(g < _N_FULL)
        def _():
            tb = g * _TRI
            pltpu.sync_copy(idx_hbm.at[pl.ds(3 * tb, _ROWS)], idx_v)
            pltpu.async_copy(nodes_hbm.at[idx_v], rows_v, gsem).wait()
            pltpu.sync_copy(rows_v.reshape(_TRI, 3, _D),
                            out_hbm.at[pl.ds(tb, _TRI)])

        return carry

    lax.fori_loop(0, _K_PER_W, chunk, 0)

    @pl.when(wid == _TAIL_WID)
    def _():
        tb = _N_FULL * _TRI
        pltpu.sync_copy(idx_hbm.at[pl.ds(3 * tb, _TAIL_ROWS)], ti_v)
        pltpu.async_copy(nodes_hbm.at[ti_v], tr_v, gsem).wait()
        pltpu.sync_copy(tr_v.reshape(_TAIL_TRI, 3, _D),
                        out_hbm.at[pl.ds(tb, _TAIL_TRI)])


@jax.jit
def kernel(nodes, triangles_indexes):
    idx = triangles_indexes.astype(jnp.int32).reshape(-1)
    mesh = plsc.VectorSubcoreMesh(core_axis_name="c", subcore_axis_name="s")
    gather = pl.kernel(
        _gather_body,
        out_type=jax.ShapeDtypeStruct((_T, 3, _D), jnp.float32),
        mesh=mesh,
        scratch_types=[
            pltpu.VMEM((_ROWS,), jnp.int32),
            pltpu.VMEM((_ROWS, _D), jnp.float32),
            pltpu.VMEM((_TAIL_ROWS,), jnp.int32),
            pltpu.VMEM((_TAIL_ROWS, _D), jnp.float32),
            pltpu.SemaphoreType.DMA,
        ],
        compiler_params=pltpu.CompilerParams(use_tc_tiling_on_sc=True),
    )
    return gather(nodes, idx)


# speedup vs baseline: 14.2023x; 2.5877x over previous
import jax
import jax.numpy as jnp
from jax import lax
from jax.experimental import pallas as pl
from jax.experimental.pallas import tpu as pltpu
from jax.experimental.pallas import tpu_sc as plsc

_N_ROWS = 600000
_D = 128
_CHUNK = 480
_N_CHUNKS = _N_ROWS // _CHUNK
_NC = 2
_NS = 16
_NW = _NC * _NS
_K_PER_W = -(-_N_CHUNKS // _NW)


def _gather_body(nodes_hbm, idx_hbm, out_hbm,
                 idx0, idx1, rows0, rows1, g0, g1, s0, s1):
    wid = lax.axis_index("s") * _NC + lax.axis_index("c")
    idx_v = (idx0, idx1)
    rows_v = (rows0, rows1)
    gsem = (g0, g1)
    ssem = (s0, s1)

    def start_gather(k, b):
        base = (wid + k * _NW) * _CHUNK
        pltpu.sync_copy(idx_hbm.at[pl.ds(base, _CHUNK)], idx_v[b])
        pltpu.async_copy(nodes_hbm.at[idx_v[b]], rows_v[b], gsem[b])

    def wait_scatter(b):
        pltpu.make_async_copy(
            rows_v[b], out_hbm.at[pl.ds(0, _CHUNK)], ssem[b]).wait()

    start_gather(0, 0)

    def step(k, b):
        g_k = wid + k * _NW
        g_n = g_k + _NW

        @pl.when(jnp.logical_and(k >= 1, g_n < _N_CHUNKS))
        def _():
            wait_scatter(1 - b)

        @pl.when(g_n < _N_CHUNKS)
        def _():
            start_gather(k + 1, 1 - b)

        @pl.when(g_k < _N_CHUNKS)
        def _():
            pltpu.make_async_copy(
                nodes_hbm.at[idx_v[b]], rows_v[b], gsem[b]).wait()
            pltpu.async_copy(
                rows_v[b], out_hbm.at[pl.ds(g_k * _CHUNK, _CHUNK)], ssem[b])

    def pair(p, carry):
        step(2 * p, 0)
        step(2 * p + 1, 1)
        return carry

    lax.fori_loop(0, _K_PER_W // 2, pair, 0)
    wait_scatter(0)
    wait_scatter(1)


@jax.jit
def kernel(nodes, triangles_indexes):
    t, v = triangles_indexes.shape
    idx = triangles_indexes.astype(jnp.int32).T.reshape(-1)
    mesh = plsc.VectorSubcoreMesh(core_axis_name="c", subcore_axis_name="s")
    gather = pl.kernel(
        _gather_body,
        out_type=jax.ShapeDtypeStruct((_N_ROWS, _D), jnp.float32),
        mesh=mesh,
        scratch_types=[
            pltpu.VMEM((_CHUNK,), jnp.int32),
            pltpu.VMEM((_CHUNK,), jnp.int32),
            pltpu.VMEM((_CHUNK, _D), jnp.float32),
            pltpu.VMEM((_CHUNK, _D), jnp.float32),
            pltpu.SemaphoreType.DMA,
            pltpu.SemaphoreType.DMA,
            pltpu.SemaphoreType.DMA,
            pltpu.SemaphoreType.DMA,
        ],
    )
    out = gather(nodes, idx)
    return out.reshape(v, t, _D).transpose(1, 0, 2)


# speedup vs baseline: 14.6674x; 1.0327x over previous
import jax
import jax.numpy as jnp
from jax import lax
from jax.experimental import pallas as pl
from jax.experimental.pallas import tpu as pltpu
from jax.experimental.pallas import tpu_sc as plsc

_N_ROWS = 600000
_D = 128
_CHUNK = 320
_N_CHUNKS = _N_ROWS // _CHUNK
_NC = 2
_NS = 16
_NW = _NC * _NS
_NBUF = 3
_K_PER_W = 3 * (-(-(-(-_N_CHUNKS // _NW)) // 3))


def _gather_body(nodes_hbm, idx_hbm, out_hbm,
                 idx0, idx1, idx2, rows0, rows1, rows2,
                 g0, g1, g2, s0, s1, s2):
    wid = lax.axis_index("s") * _NC + lax.axis_index("c")
    idx_v = (idx0, idx1, idx2)
    rows_v = (rows0, rows1, rows2)
    gsem = (g0, g1, g2)
    ssem = (s0, s1, s2)

    def start_gather(k, b):
        base = (wid + k * _NW) * _CHUNK
        pltpu.sync_copy(idx_hbm.at[pl.ds(base, _CHUNK)], idx_v[b])
        pltpu.async_copy(nodes_hbm.at[idx_v[b]], rows_v[b], gsem[b])

    def wait_scatter(b):
        pltpu.make_async_copy(
            rows_v[b], out_hbm.at[pl.ds(0, _CHUNK)], ssem[b]).wait()

    start_gather(0, 0)

    def step(k, b, b1):
        g_k = wid + k * _NW
        g_n = g_k + _NW

        @pl.when(jnp.logical_and(k >= 2, g_n < _N_CHUNKS))
        def _():
            wait_scatter(b1)

        @pl.when(g_n < _N_CHUNKS)
        def _():
            start_gather(k + 1, b1)

        @pl.when(g_k < _N_CHUNKS)
        def _():
            pltpu.make_async_copy(
                nodes_hbm.at[idx_v[b]], rows_v[b], gsem[b]).wait()
            pltpu.async_copy(
                rows_v[b], out_hbm.at[pl.ds(g_k * _CHUNK, _CHUNK)], ssem[b])

    def triple(p, carry):
        k0 = 3 * p
        step(k0, 0, 1)
        step(k0 + 1, 1, 2)
        step(k0 + 2, 2, 0)
        return carry

    lax.fori_loop(0, _K_PER_W // 3, triple, 0)
    wait_scatter(0)
    wait_scatter(1)
    wait_scatter(2)


@jax.jit
def kernel(nodes, triangles_indexes):
    t, v = triangles_indexes.shape
    idx = triangles_indexes.astype(jnp.int32).T.reshape(-1)
    mesh = plsc.VectorSubcoreMesh(core_axis_name="c", subcore_axis_name="s")
    gather = pl.kernel(
        _gather_body,
        out_type=jax.ShapeDtypeStruct((_N_ROWS, _D), jnp.float32),
        mesh=mesh,
        scratch_types=[
            pltpu.VMEM((_CHUNK,), jnp.int32),
            pltpu.VMEM((_CHUNK,), jnp.int32),
            pltpu.VMEM((_CHUNK,), jnp.int32),
            pltpu.VMEM((_CHUNK, _D), jnp.float32),
            pltpu.VMEM((_CHUNK, _D), jnp.float32),
            pltpu.VMEM((_CHUNK, _D), jnp.float32),
            pltpu.SemaphoreType.DMA,
            pltpu.SemaphoreType.DMA,
            pltpu.SemaphoreType.DMA,
            pltpu.SemaphoreType.DMA,
            pltpu.SemaphoreType.DMA,
            pltpu.SemaphoreType.DMA,
        ],
    )
    out = gather(nodes, idx)
    return out.reshape(v, t, _D).transpose(1, 0, 2)
